# Initial kernel scaffold; baseline (speedup 1.0000x reference)
import jax
import jax.numpy as jnp
from jax.experimental import pallas as pl


def kernel(x, edge_index, Ws1, Wn1, b1, Ws2, Wn2, b2, Ws3, Wn3, b3, Ws4, Wn4, b4):
    raise NotImplementedError("write your pallas kernel here")


# speedup vs baseline: 8.3378x; 8.3378x over previous
import functools

import jax
import jax.numpy as jnp
from jax import lax
from jax.experimental import pallas as pl
from jax.experimental.pallas import tpu as pltpu
from jax.experimental.pallas import tpu_sc as plsc

N = 10000
E = 320000
D = 128

NC = 2
NS = 16
NW = NC * NS
K = 128
C = -(-E // (NW * K))
EP = NW * K * C
NP = 10240
SL = NP // NS

_mesh = plsc.VectorSubcoreMesh(core_axis_name="c", subcore_axis_name="s")


@functools.partial(
    pl.kernel,
    out_type=jax.ShapeDtypeStruct((NC, NP), jnp.float32),
    mesh=_mesh,
    scratch_types=[
        pltpu.VMEM((C, K), jnp.int32),
        pltpu.VMEM((K,), jnp.float32),
        pltpu.VMEM_SHARED((NP,), jnp.float32),
    ],
)
def _deg_kernel(dst_hbm, zeros1_hbm, ones_hbm, degp_hbm, dst_v, ones_v, acc):
    c = lax.axis_index("c")
    s = lax.axis_index("s")
    wid = s * NC + c
    pltpu.sync_copy(dst_hbm.at[wid], dst_v)
    pltpu.sync_copy(ones_hbm, ones_v)
    pltpu.sync_copy(zeros1_hbm.at[pl.ds(s * SL, SL)], acc.at[pl.ds(s * SL, SL)])
    plsc.subcore_barrier()

    def body(j, carry):
        pltpu.sync_copy(ones_v, acc.at[dst_v.at[j]], add=True)
        return carry

    lax.fori_loop(0, C, body, 0)
    plsc.subcore_barrier()
    pltpu.sync_copy(acc.at[pl.ds(s * SL, SL)], degp_hbm.at[c, pl.ds(s * SL, SL)])


@functools.partial(
    pl.kernel,
    out_type=jax.ShapeDtypeStruct((NC, NP, D), jnp.float32),
    mesh=_mesh,
    scratch_types=[
        pltpu.VMEM((C, K), jnp.int32),
        pltpu.VMEM((C, K), jnp.int32),
        pltpu.VMEM((K, D), jnp.float32),
        pltpu.VMEM_SHARED((NP, D), jnp.float32),
        pltpu.SemaphoreType.DMA,
    ],
)
def _seg_kernel(y_hbm, src_hbm, dst_hbm, zeros2_hbm, sp_hbm,
                src_v, dst_v, rows_v, acc, sem):
    c = lax.axis_index("c")
    s = lax.axis_index("s")
    wid = s * NC + c
    pltpu.sync_copy(src_hbm.at[wid], src_v)
    pltpu.sync_copy(dst_hbm.at[wid], dst_v)
    pltpu.sync_copy(zeros2_hbm.at[pl.ds(s * SL, SL)], acc.at[pl.ds(s * SL, SL)])
    plsc.subcore_barrier()

    def body(j, carry):
        pltpu.async_copy(y_hbm.at[src_v.at[j]], rows_v, sem).wait()
        pltpu.sync_copy(rows_v, acc.at[dst_v.at[j]], add=True)
        return carry

    lax.fori_loop(0, C, body, 0)
    plsc.subcore_barrier()
    pltpu.sync_copy(acc.at[pl.ds(s * SL, SL)], sp_hbm.at[c, pl.ds(s * SL, SL)])


_BN = 2000
_G = N // _BN
_P32 = jnp.float32


def _row_spec(w):
    return pl.BlockSpec((_BN, w), lambda i: (i, 0))


def _full_spec(r, w):
    return pl.BlockSpec((r, w), lambda i: (0, 0))


def _invd_body(da_ref, db_ref, o_ref):
    o_ref[...] = 1.0 / jnp.maximum(da_ref[...] + db_ref[...], 1.0)


def _invd(da, db):
    return pl.pallas_call(
        _invd_body,
        grid=(_G,),
        in_specs=[_row_spec(1), _row_spec(1)],
        out_specs=_row_spec(1),
        out_shape=jax.ShapeDtypeStruct((N, 1), _P32),
    )(da, db)


def _gnb_body(h_ref, sa_ref, sb_ref, iv_ref, ws_ref, wn_ref, b_ref, o_ref):
    agg = (sa_ref[...] + sb_ref[...]) * iv_ref[...]
    o_ref[...] = jnp.maximum(
        jnp.dot(h_ref[...], ws_ref[...]) + jnp.dot(agg, wn_ref[...])
        + b_ref[...], 0.0)


def _gnb(h, sa, sb, iv, ws, wn, b):
    return pl.pallas_call(
        _gnb_body,
        grid=(_G,),
        in_specs=[_row_spec(D), _row_spec(D), _row_spec(D), _row_spec(1),
                  _full_spec(D, D), _full_spec(D, D), _full_spec(1, D)],
        out_specs=_row_spec(D),
        out_shape=jax.ShapeDtypeStruct((N, D), _P32),
    )(h, sa, sb, iv, ws, wn, b)


def _gnb4_body(h_ref, sa_ref, sb_ref, iv_ref, ws_ref, wn_ref, b_ref, o_ref):
    agg = (sa_ref[...] + sb_ref[...]) * iv_ref[...]
    o_ref[...] = jax.nn.sigmoid(jnp.maximum(
        jnp.dot(h_ref[...], ws_ref[...]) + jnp.dot(agg, wn_ref[...])
        + b_ref[...], 0.0))


def _gnb4(h, sa, sb, iv, ws, wn, b):
    return pl.pallas_call(
        _gnb4_body,
        grid=(_G,),
        in_specs=[_row_spec(D), _row_spec(D), _row_spec(D), _row_spec(1),
                  _full_spec(D, 1), _full_spec(D, 1), _full_spec(1, 1)],
        out_specs=_row_spec(1),
        out_shape=jax.ShapeDtypeStruct((N, 1), _P32),
    )(h, sa, sb, iv, ws, wn, b)


def kernel(x, edge_index, Ws1, Wn1, b1, Ws2, Wn2, b2, Ws3, Wn3, b3,
           Ws4, Wn4, b4):
    src = edge_index[0]
    dst = edge_index[1]
    pad = EP - E
    ar = lax.iota(jnp.int32, pad)
    src_p = jnp.concatenate([src, ar % N]).reshape(NW, C, K)
    dst_p = jnp.concatenate([dst, N + (ar % (NP - N))]).reshape(NW, C, K)

    zeros1 = jnp.zeros((NP,), _P32)
    zeros2 = jnp.zeros((NP, D), _P32)
    ones = jnp.ones((K,), _P32)

    degp = _deg_kernel(dst_p, zeros1, ones)
    iv = _invd(degp[0, :N, None], degp[1, :N, None])

    ag = _seg_kernel(x, src_p, dst_p, zeros2)
    h1 = _gnb(x, ag[0, :N], ag[1, :N], iv, Ws1, Wn1, b1[None, :])
    ag = _seg_kernel(h1, src_p, dst_p, zeros2)
    h2 = _gnb(h1, ag[0, :N], ag[1, :N], iv, Ws2, Wn2, b2[None, :])
    ag = _seg_kernel(h2, src_p, dst_p, zeros2)
    h3 = _gnb(h2, ag[0, :N], ag[1, :N], iv, Ws3, Wn3, b3[None, :])
    ag = _seg_kernel(h3, src_p, dst_p, zeros2)
    return _gnb4(h3, ag[0, :N], ag[1, :N], iv, Ws4, Wn4, b4[None, :])


# speedup vs baseline: 12.2622x; 1.4707x over previous
import functools

import jax
import jax.numpy as jnp
from jax import lax
from jax.experimental import pallas as pl
from jax.experimental.pallas import tpu as pltpu
from jax.experimental.pallas import tpu_sc as plsc

N = 10000
E = 320000
D = 128

NC = 2
NS = 16
NW = NC * NS
K = 128
C = 80
EP = NW * K * C
NP = 10240
SL = NP // NS

_mesh = plsc.VectorSubcoreMesh(core_axis_name="c", subcore_axis_name="s")


@functools.partial(
    pl.kernel,
    out_type=jax.ShapeDtypeStruct((NC, NP), jnp.float32),
    mesh=_mesh,
    scratch_types=[
        pltpu.VMEM((C, K), jnp.int32),
        pltpu.VMEM((K,), jnp.float32),
        pltpu.VMEM_SHARED((NP,), jnp.float32),
    ],
)
def _deg_kernel(dst_hbm, zeros1_hbm, ones_hbm, degp_hbm, dst_v, ones_v, acc):
    c = lax.axis_index("c")
    s = lax.axis_index("s")
    wid = s * NC + c
    pltpu.sync_copy(dst_hbm.at[wid], dst_v)
    pltpu.sync_copy(ones_hbm, ones_v)
    pltpu.sync_copy(zeros1_hbm.at[pl.ds(s * SL, SL)], acc.at[pl.ds(s * SL, SL)])
    plsc.subcore_barrier()

    def body(j, carry):
        pltpu.sync_copy(ones_v, acc.at[dst_v.at[j]], add=True)
        return carry

    lax.fori_loop(0, C, body, 0)
    plsc.subcore_barrier()
    pltpu.sync_copy(acc.at[pl.ds(s * SL, SL)], degp_hbm.at[c, pl.ds(s * SL, SL)])


@functools.partial(
    pl.kernel,
    out_type=jax.ShapeDtypeStruct((NC, NP, D), jnp.float32),
    mesh=_mesh,
    scratch_types=[
        pltpu.VMEM((C, K), jnp.int32),
        pltpu.VMEM((2, K), jnp.int32),
        pltpu.VMEM((K, D), jnp.float32),
        pltpu.VMEM((K, D), jnp.float32),
        pltpu.VMEM_SHARED((NP, D), jnp.float32),
        pltpu.SemaphoreType.DMA,
        pltpu.SemaphoreType.DMA,
        pltpu.SemaphoreType.DMA,
        pltpu.SemaphoreType.DMA,
    ],
)
def _seg_kernel(y_hbm, src_hbm, dst_hbm, zeros2_hbm, sp_hbm,
                src_v, dstb, rows0, rows1, acc, sem0, sem1, semd0, semd1):
    c = lax.axis_index("c")
    s = lax.axis_index("s")
    wid = s * NC + c
    pltpu.sync_copy(src_hbm.at[wid], src_v)
    pltpu.sync_copy(zeros2_hbm.at[pl.ds(s * SL, SL)], acc.at[pl.ds(s * SL, SL)])
    plsc.subcore_barrier()

    def gather(j, rows, sem):
        return pltpu.make_async_copy(y_hbm.at[src_v.at[j]], rows, sem)

    def dst_fetch(j, b, sem):
        return pltpu.make_async_copy(dst_hbm.at[wid, j], dstb.at[b], sem)

    gather(0, rows0, sem0).start()
    dst_fetch(0, 0, semd0).start()
    dst_fetch(1, 1, semd1).start()

    def body(i, carry):
        j0 = 2 * i
        j1 = 2 * i + 1
        j2 = jnp.minimum(j1 + 1, C - 1)
        j3 = jnp.minimum(j1 + 2, C - 1)
        gather(j1, rows1, sem1).start()
        gather(j0, rows0, sem0).wait()
        dst_fetch(j0, 0, semd0).wait()
        pltpu.sync_copy(rows0, acc.at[dstb.at[0]], add=True)
        dst_fetch(j2, 0, semd0).start()
        gather(j2, rows0, sem0).start()
        gather(j1, rows1, sem1).wait()
        dst_fetch(j1, 1, semd1).wait()
        pltpu.sync_copy(rows1, acc.at[dstb.at[1]], add=True)
        dst_fetch(j3, 1, semd1).start()
        return carry

    lax.fori_loop(0, C // 2, body, 0)
    gather(C - 1, rows0, sem0).wait()
    dst_fetch(C - 1, 0, semd0).wait()
    dst_fetch(C - 1, 1, semd1).wait()
    plsc.subcore_barrier()
    pltpu.sync_copy(acc.at[pl.ds(s * SL, SL)], sp_hbm.at[c, pl.ds(s * SL, SL)])


_BN = 2000
_G = N // _BN
_P32 = jnp.float32


def _row_spec(w):
    return pl.BlockSpec((_BN, w), lambda i: (i, 0))


def _full_spec(r, w):
    return pl.BlockSpec((r, w), lambda i: (0, 0))


def _gnb_body(h_ref, sa_ref, sb_ref, da_ref, db_ref, ws_ref, wn_ref, b_ref,
              o_ref):
    iv = 1.0 / jnp.maximum(da_ref[...] + db_ref[...], 1.0)
    agg = (sa_ref[...] + sb_ref[...]) * iv
    o_ref[...] = jnp.maximum(
        jnp.dot(h_ref[...], ws_ref[...]) + jnp.dot(agg, wn_ref[...])
        + b_ref[...], 0.0)


def _gnb(h, sa, sb, da, db, ws, wn, b):
    return pl.pallas_call(
        _gnb_body,
        grid=(_G,),
        in_specs=[_row_spec(D), _row_spec(D), _row_spec(D), _row_spec(1),
                  _row_spec(1), _full_spec(D, D), _full_spec(D, D),
                  _full_spec(1, D)],
        out_specs=_row_spec(D),
        out_shape=jax.ShapeDtypeStruct((N, D), _P32),
    )(h, sa, sb, da, db, ws, wn, b)


def _gnb4_body(h_ref, sa_ref, sb_ref, da_ref, db_ref, ws_ref, wn_ref, b_ref,
               o_ref):
    iv = 1.0 / jnp.maximum(da_ref[...] + db_ref[...], 1.0)
    agg = (sa_ref[...] + sb_ref[...]) * iv
    o_ref[...] = jax.nn.sigmoid(jnp.maximum(
        jnp.dot(h_ref[...], ws_ref[...]) + jnp.dot(agg, wn_ref[...])
        + b_ref[...], 0.0))


def _gnb4(h, sa, sb, da, db, ws, wn, b):
    return pl.pallas_call(
        _gnb4_body,
        grid=(_G,),
        in_specs=[_row_spec(D), _row_spec(D), _row_spec(D), _row_spec(1),
                  _row_spec(1), _full_spec(D, 1), _full_spec(D, 1),
                  _full_spec(1, 1)],
        out_specs=_row_spec(1),
        out_shape=jax.ShapeDtypeStruct((N, 1), _P32),
    )(h, sa, sb, da, db, ws, wn, b)


def kernel(x, edge_index, Ws1, Wn1, b1, Ws2, Wn2, b2, Ws3, Wn3, b3,
           Ws4, Wn4, b4):
    src = edge_index[0]
    dst = edge_index[1]
    pad = EP - E
    ar = lax.iota(jnp.int32, pad)
    src_p = jnp.concatenate([src, ar % N]).reshape(NW, C, K)
    dst_p = jnp.concatenate([dst, N + (ar % (NP - N))]).reshape(NW, C, K)

    zeros1 = jnp.zeros((NP,), _P32)
    zeros2 = jnp.zeros((NP, D), _P32)
    ones = jnp.ones((K,), _P32)

    degp = _deg_kernel(dst_p, zeros1, ones)
    da = degp[0, :N, None]
    db = degp[1, :N, None]

    ag = _seg_kernel(x, src_p, dst_p, zeros2)
    h1 = _gnb(x, ag[0, :N], ag[1, :N], da, db, Ws1, Wn1, b1[None, :])
    ag = _seg_kernel(h1, src_p, dst_p, zeros2)
    h2 = _gnb(h1, ag[0, :N], ag[1, :N], da, db, Ws2, Wn2, b2[None, :])
    ag = _seg_kernel(h2, src_p, dst_p, zeros2)
    h3 = _gnb(h2, ag[0, :N], ag[1, :N], da, db, Ws3, Wn3, b3[None, :])
    ag = _seg_kernel(h3, src_p, dst_p, zeros2)
    return _gnb4(h3, ag[0, :N], ag[1, :N], da, db, Ws4, Wn4, b4[None, :])
